# baseline (speedup 1.0000x reference)
import functools

import jax
import jax.numpy as jnp
from jax import lax
from jax.experimental import pallas as pl
from jax.experimental.pallas import tpu as pltpu

KSIZE = 5
PAD = KSIZE // 2


def _round_up(v, m):
    return (v + m - 1) // m * m


def _reduce_kernel(x_ref, avg_ref, max_ref, *, BN, C, HW, CHUNK):
    inv_c = jnp.float32(1.0 / C)
    n_full = C // CHUNK
    rem = C - n_full * CHUNK
    for b in range(BN):
        base = b * C
        s0 = jnp.zeros((CHUNK, HW), jnp.float32)
        m0 = jnp.full((CHUNK, HW), -jnp.inf, jnp.float32)

        def body(i, carry, base=base):
            s, m = carry
            blk = x_ref[pl.ds(base + i * CHUNK, CHUNK), :].astype(jnp.float32)
            return s + blk, jnp.maximum(m, blk)

        s, m = lax.fori_loop(0, n_full, body, (s0, m0), unroll=8)
        if rem:
            blk = x_ref[pl.ds(base + n_full * CHUNK, rem), :].astype(jnp.float32)
            s = s.at[:rem].add(blk)
            m = m.at[:rem].max(blk)
        avg_ref[pl.ds(b, 1), :] = jnp.sum(s, axis=0, keepdims=True) * inv_c
        max_ref[pl.ds(b, 1), :] = jnp.max(m, axis=0, keepdims=True)


def _conv_kernel(w_ref, avg_ref, max_ref, o_ref, pa_ref, pm_ref,
                 *, N, H, W, HALF, PADS):
    HW = H * W
    i = pl.program_id(0)
    tail = pa_ref.shape[0] - PADS - HW

    pa_ref[pl.ds(PADS, HW), :] = jnp.transpose(avg_ref[...], (1, 0))
    pm_ref[pl.ds(PADS, HW), :] = jnp.transpose(max_ref[...], (1, 0))
    pa_ref[pl.ds(0, PADS), :] = jnp.zeros((PADS, N), jnp.float32)
    pm_ref[pl.ds(0, PADS), :] = jnp.zeros((PADS, N), jnp.float32)
    pa_ref[pl.ds(PADS + HW, tail), :] = jnp.zeros((tail, N), jnp.float32)
    pm_ref[pl.ds(PADS + HW, tail), :] = jnp.zeros((tail, N), jnp.float32)

    r = lax.broadcasted_iota(jnp.int32, (HALF, N), 0)
    px = (r + i * HALF) % W
    base0 = PADS + i * HALF

    acc = jnp.zeros((HALF, N), jnp.float32)
    for dx in range(-PAD, PAD + 1):
        valid = jnp.logical_and(px + dx >= 0, px + dx < W)
        acc_dx = jnp.zeros((HALF, N), jnp.float32)
        for dy in range(-PAD, PAD + 1):
            s = dy * W + dx
            wa = w_ref[(dy + PAD) * KSIZE + (dx + PAD)]
            wm = w_ref[KSIZE * KSIZE + (dy + PAD) * KSIZE + (dx + PAD)]
            acc_dx = acc_dx + pa_ref[pl.ds(base0 + s, HALF), :] * wa
            acc_dx = acc_dx + pm_ref[pl.ds(base0 + s, HALF), :] * wm
        acc = acc + jnp.where(valid, acc_dx, jnp.float32(0.0))

    res = jax.nn.sigmoid(acc)
    o_ref[...] = jnp.transpose(res, (1, 0)).astype(o_ref.dtype)


def kernel(x, w):
    N, C, H, W = x.shape
    HW = H * W
    x2 = x.reshape(N * C, HW)
    w_flat = w.reshape(-1).astype(jnp.float32)

    BN = 1
    for cand in (8, 4, 2, 1):
        if N % cand == 0:
            BN = cand
            break
    CHUNK = 8
    red_fn = functools.partial(_reduce_kernel, BN=BN, C=C, HW=HW, CHUNK=CHUNK)
    avg, mx = pl.pallas_call(
        red_fn,
        out_shape=(jax.ShapeDtypeStruct((N, HW), jnp.float32),
                   jax.ShapeDtypeStruct((N, HW), jnp.float32)),
        grid=(N // BN,),
        in_specs=[pl.BlockSpec((BN * C, HW), lambda i: (i, 0))],
        out_specs=(pl.BlockSpec((BN, HW), lambda i: (i, 0)),
                   pl.BlockSpec((BN, HW), lambda i: (i, 0))),
        compiler_params=pltpu.CompilerParams(
            dimension_semantics=("parallel",),
            vmem_limit_bytes=48 * 1024 * 1024,
        ),
    )(x2)

    n_half = 2 if HW % 2 == 0 else 1
    HALF = HW // n_half
    PADS = _round_up(PAD * W + PAD, 8)
    rows = _round_up(HW + 2 * PADS, 8)
    conv_fn = functools.partial(_conv_kernel, N=N, H=H, W=W, HALF=HALF,
                                PADS=PADS)
    out = pl.pallas_call(
        conv_fn,
        out_shape=jax.ShapeDtypeStruct((N, HW), x.dtype),
        grid=(n_half,),
        in_specs=[
            pl.BlockSpec(memory_space=pltpu.MemorySpace.SMEM),
            pl.BlockSpec((N, HW), lambda i: (0, 0)),
            pl.BlockSpec((N, HW), lambda i: (0, 0)),
        ],
        out_specs=pl.BlockSpec((N, HALF), lambda i: (0, i)),
        scratch_shapes=[
            pltpu.VMEM((rows, N), jnp.float32),
            pltpu.VMEM((rows, N), jnp.float32),
        ],
        compiler_params=pltpu.CompilerParams(
            dimension_semantics=("parallel",),
            vmem_limit_bytes=48 * 1024 * 1024,
        ),
    )(w_flat, avg, mx)

    return out.reshape(N, 1, H, W)


# speedup vs baseline: 6.4747x; 6.4747x over previous
import functools

import jax
import jax.numpy as jnp
from jax import lax
from jax.experimental import pallas as pl
from jax.experimental.pallas import tpu as pltpu

KSIZE = 5
PAD = KSIZE // 2


def _round_up(v, m):
    return (v + m - 1) // m * m


def _reduce_kernel(x_ref, avg_ref, max_ref, *, BN, C, HW):
    inv_c = jnp.float32(1.0 / C)
    for b in range(BN):
        v = x_ref[b]
        if C % 128 == 0 and C > 128:
            s = v[:, :128]
            m = v[:, :128]
            for g in range(1, C // 128):
                blk = v[:, g * 128:(g + 1) * 128]
                s = s + blk
                m = jnp.maximum(m, blk)
        else:
            s = v
            m = v
        col_s = jnp.sum(s, axis=-1, keepdims=True) * inv_c
        col_m = jnp.max(m, axis=-1, keepdims=True)
        avg_ref[pl.ds(b, 1), :] = jnp.transpose(col_s, (1, 0))
        max_ref[pl.ds(b, 1), :] = jnp.transpose(col_m, (1, 0))


def _conv_kernel(w_ref, avg_ref, max_ref, o_ref, pa_ref, pm_ref,
                 *, N, H, W, HALF, PADS):
    HW = H * W
    i = pl.program_id(0)
    tail = pa_ref.shape[0] - PADS - HW

    pa_ref[pl.ds(PADS, HW), :] = jnp.transpose(avg_ref[...], (1, 0))
    pm_ref[pl.ds(PADS, HW), :] = jnp.transpose(max_ref[...], (1, 0))
    pa_ref[pl.ds(0, PADS), :] = jnp.zeros((PADS, N), jnp.float32)
    pm_ref[pl.ds(0, PADS), :] = jnp.zeros((PADS, N), jnp.float32)
    pa_ref[pl.ds(PADS + HW, tail), :] = jnp.zeros((tail, N), jnp.float32)
    pm_ref[pl.ds(PADS + HW, tail), :] = jnp.zeros((tail, N), jnp.float32)

    r = lax.broadcasted_iota(jnp.int32, (HALF, N), 0)
    px = (r + i * HALF) % W
    base0 = PADS + i * HALF

    acc = jnp.zeros((HALF, N), jnp.float32)
    for dx in range(-PAD, PAD + 1):
        valid = jnp.logical_and(px + dx >= 0, px + dx < W)
        acc_dx = jnp.zeros((HALF, N), jnp.float32)
        for dy in range(-PAD, PAD + 1):
            s = dy * W + dx
            wa = w_ref[(dy + PAD) * KSIZE + (dx + PAD)]
            wm = w_ref[KSIZE * KSIZE + (dy + PAD) * KSIZE + (dx + PAD)]
            acc_dx = acc_dx + pa_ref[pl.ds(base0 + s, HALF), :] * wa
            acc_dx = acc_dx + pm_ref[pl.ds(base0 + s, HALF), :] * wm
        acc = acc + jnp.where(valid, acc_dx, jnp.float32(0.0))

    o_ref[...] = jax.nn.sigmoid(acc).astype(o_ref.dtype)


def kernel(x, w):
    N, C, H, W = x.shape
    HW = H * W
    xt = x.transpose(0, 2, 3, 1).reshape(N, HW, C)
    w_flat = w.reshape(-1).astype(jnp.float32)

    BN = 1
    for cand in (8, 4, 2, 1):
        if N % cand == 0:
            BN = cand
            break
    red_fn = functools.partial(_reduce_kernel, BN=BN, C=C, HW=HW)
    avg, mx = pl.pallas_call(
        red_fn,
        out_shape=(jax.ShapeDtypeStruct((N, HW), jnp.float32),
                   jax.ShapeDtypeStruct((N, HW), jnp.float32)),
        grid=(N // BN,),
        in_specs=[pl.BlockSpec((BN, HW, C), lambda i: (i, 0, 0))],
        out_specs=(pl.BlockSpec((BN, HW), lambda i: (i, 0)),
                   pl.BlockSpec((BN, HW), lambda i: (i, 0))),
        compiler_params=pltpu.CompilerParams(
            dimension_semantics=("parallel",),
            vmem_limit_bytes=48 * 1024 * 1024,
        ),
    )(xt)

    n_half = 2 if HW % 2 == 0 else 1
    HALF = HW // n_half
    PADS = _round_up(PAD * W + PAD, 8)
    rows = _round_up(HW + 2 * PADS, 8)
    conv_fn = functools.partial(_conv_kernel, N=N, H=H, W=W, HALF=HALF,
                                PADS=PADS)
    out_t = pl.pallas_call(
        conv_fn,
        out_shape=jax.ShapeDtypeStruct((HW, N), x.dtype),
        grid=(n_half,),
        in_specs=[
            pl.BlockSpec(memory_space=pltpu.MemorySpace.SMEM),
            pl.BlockSpec((N, HW), lambda i: (0, 0)),
            pl.BlockSpec((N, HW), lambda i: (0, 0)),
        ],
        out_specs=pl.BlockSpec((HALF, N), lambda i: (i, 0)),
        scratch_shapes=[
            pltpu.VMEM((rows, N), jnp.float32),
            pltpu.VMEM((rows, N), jnp.float32),
        ],
        compiler_params=pltpu.CompilerParams(
            dimension_semantics=("parallel",),
            vmem_limit_bytes=48 * 1024 * 1024,
        ),
    )(w_flat, avg, mx)

    return out_t.transpose(1, 0).reshape(N, 1, H, W)


# speedup vs baseline: 7.2000x; 1.1120x over previous
import functools

import jax
import jax.numpy as jnp
from jax import lax
from jax.experimental import pallas as pl
from jax.experimental.pallas import tpu as pltpu

KSIZE = 5
PAD = KSIZE // 2


def _round_up(v, m):
    return (v + m - 1) // m * m


def _reduce_kernel(x_ref, avg_ref, max_ref, *, BN, C, HW):
    inv_c = jnp.float32(1.0 / C)
    for b in range(BN):
        v = x_ref[b]
        if C % 128 == 0 and C > 128:
            s = v[:, :128]
            m = v[:, :128]
            for g in range(1, C // 128):
                blk = v[:, g * 128:(g + 1) * 128]
                s = s + blk
                m = jnp.maximum(m, blk)
        else:
            s = v
            m = v
        col_s = jnp.sum(s, axis=-1, keepdims=True) * inv_c
        col_m = jnp.max(m, axis=-1, keepdims=True)
        avg_ref[pl.ds(b, 1), :] = jnp.transpose(col_s, (1, 0))
        max_ref[pl.ds(b, 1), :] = jnp.transpose(col_m, (1, 0))


def _conv_kernel(w_ref, avg_ref, max_ref, o_ref, pa_ref, pm_ref,
                 *, N, H, W, HALF, PADS):
    HW = H * W
    i = pl.program_id(0)
    tail = pa_ref.shape[0] - PADS - HW

    pa_ref[pl.ds(PADS, HW), :] = jnp.transpose(avg_ref[...], (1, 0))
    pm_ref[pl.ds(PADS, HW), :] = jnp.transpose(max_ref[...], (1, 0))
    pa_ref[pl.ds(0, PADS), :] = jnp.zeros((PADS, N), jnp.float32)
    pm_ref[pl.ds(0, PADS), :] = jnp.zeros((PADS, N), jnp.float32)
    pa_ref[pl.ds(PADS + HW, tail), :] = jnp.zeros((tail, N), jnp.float32)
    pm_ref[pl.ds(PADS + HW, tail), :] = jnp.zeros((tail, N), jnp.float32)

    r = lax.broadcasted_iota(jnp.int32, (HALF, N), 0)
    px = (r + i * HALF) % W
    base0 = PADS + i * HALF

    acc = jnp.zeros((HALF, N), jnp.float32)
    for dx in range(-PAD, PAD + 1):
        valid = jnp.logical_and(px + dx >= 0, px + dx < W)
        acc_dx = jnp.zeros((HALF, N), jnp.float32)
        for dy in range(-PAD, PAD + 1):
            s = dy * W + dx
            wa = w_ref[(dy + PAD) * KSIZE + (dx + PAD)]
            wm = w_ref[KSIZE * KSIZE + (dy + PAD) * KSIZE + (dx + PAD)]
            acc_dx = acc_dx + pa_ref[pl.ds(base0 + s, HALF), :] * wa
            acc_dx = acc_dx + pm_ref[pl.ds(base0 + s, HALF), :] * wm
        acc = acc + jnp.where(valid, acc_dx, jnp.float32(0.0))

    o_ref[...] = jax.nn.sigmoid(acc).astype(o_ref.dtype)


def kernel(x, w):
    N, C, H, W = x.shape
    HW = H * W
    xt = x.transpose(0, 2, 3, 1).reshape(N, HW, C)
    w_flat = w.reshape(-1).astype(jnp.float32)

    BN = 1
    for cand in (16, 8, 4, 2, 1):
        if N % cand == 0:
            BN = cand
            break
    red_fn = functools.partial(_reduce_kernel, BN=BN, C=C, HW=HW)
    avg, mx = pl.pallas_call(
        red_fn,
        out_shape=(jax.ShapeDtypeStruct((N, HW), jnp.float32),
                   jax.ShapeDtypeStruct((N, HW), jnp.float32)),
        grid=(N // BN,),
        in_specs=[pl.BlockSpec((BN, HW, C), lambda i: (i, 0, 0))],
        out_specs=(pl.BlockSpec((BN, HW), lambda i: (i, 0)),
                   pl.BlockSpec((BN, HW), lambda i: (i, 0))),
        compiler_params=pltpu.CompilerParams(
            dimension_semantics=("parallel",),
            vmem_limit_bytes=48 * 1024 * 1024,
        ),
    )(xt)

    n_half = 2 if HW % 2 == 0 else 1
    HALF = HW // n_half
    PADS = _round_up(PAD * W + PAD, 8)
    rows = _round_up(HW + 2 * PADS, 8)
    conv_fn = functools.partial(_conv_kernel, N=N, H=H, W=W, HALF=HALF,
                                PADS=PADS)
    out_t = pl.pallas_call(
        conv_fn,
        out_shape=jax.ShapeDtypeStruct((HW, N), x.dtype),
        grid=(n_half,),
        in_specs=[
            pl.BlockSpec(memory_space=pltpu.MemorySpace.SMEM),
            pl.BlockSpec((N, HW), lambda i: (0, 0)),
            pl.BlockSpec((N, HW), lambda i: (0, 0)),
        ],
        out_specs=pl.BlockSpec((HALF, N), lambda i: (i, 0)),
        scratch_shapes=[
            pltpu.VMEM((rows, N), jnp.float32),
            pltpu.VMEM((rows, N), jnp.float32),
        ],
        compiler_params=pltpu.CompilerParams(
            dimension_semantics=("parallel",),
            vmem_limit_bytes=48 * 1024 * 1024,
        ),
    )(w_flat, avg, mx)

    return out_t.transpose(1, 0).reshape(N, 1, H, W)


# speedup vs baseline: 7.2408x; 1.0057x over previous
import functools

import jax
import jax.numpy as jnp
from jax import lax
from jax.experimental import pallas as pl
from jax.experimental.pallas import tpu as pltpu

KSIZE = 5
PAD = KSIZE // 2


def _round_up(v, m):
    return (v + m - 1) // m * m


def _reduce_kernel(x_ref, avg_ref, max_ref, *, BN, C, HW):
    inv_c = jnp.float32(1.0 / C)
    for b in range(BN):
        v = x_ref[b]
        if C % 128 == 0 and C > 128:
            s = v[:, :128]
            m = v[:, :128]
            for g in range(1, C // 128):
                blk = v[:, g * 128:(g + 1) * 128]
                s = s + blk
                m = jnp.maximum(m, blk)
        else:
            s = v
            m = v
        col_s = jnp.sum(s, axis=-1, keepdims=True) * inv_c
        col_m = jnp.max(m, axis=-1, keepdims=True)
        avg_ref[pl.ds(b, 1), :] = jnp.transpose(col_s, (1, 0))
        max_ref[pl.ds(b, 1), :] = jnp.transpose(col_m, (1, 0))


def _conv_kernel(w_ref, avg_ref, max_ref, o_ref, pa_ref, pm_ref,
                 *, N, H, W, HALF, PADS):
    HW = H * W
    i = pl.program_id(0)
    tail = pa_ref.shape[0] - PADS - HW

    pa_ref[pl.ds(PADS, HW), :] = jnp.transpose(avg_ref[...], (1, 0))
    pm_ref[pl.ds(PADS, HW), :] = jnp.transpose(max_ref[...], (1, 0))
    pa_ref[pl.ds(0, PADS), :] = jnp.zeros((PADS, N), jnp.float32)
    pm_ref[pl.ds(0, PADS), :] = jnp.zeros((PADS, N), jnp.float32)
    pa_ref[pl.ds(PADS + HW, tail), :] = jnp.zeros((tail, N), jnp.float32)
    pm_ref[pl.ds(PADS + HW, tail), :] = jnp.zeros((tail, N), jnp.float32)

    r = lax.broadcasted_iota(jnp.int32, (HALF, N), 0)
    px = (r + i * HALF) % W
    base0 = PADS + i * HALF

    acc = jnp.zeros((HALF, N), jnp.float32)
    for dx in range(-PAD, PAD + 1):
        valid = jnp.logical_and(px + dx >= 0, px + dx < W)
        acc_dx = jnp.zeros((HALF, N), jnp.float32)
        for dy in range(-PAD, PAD + 1):
            s = dy * W + dx
            wa = w_ref[(dy + PAD) * KSIZE + (dx + PAD)]
            wm = w_ref[KSIZE * KSIZE + (dy + PAD) * KSIZE + (dx + PAD)]
            acc_dx = acc_dx + pa_ref[pl.ds(base0 + s, HALF), :] * wa
            acc_dx = acc_dx + pm_ref[pl.ds(base0 + s, HALF), :] * wm
        acc = acc + jnp.where(valid, acc_dx, jnp.float32(0.0))

    o_ref[...] = jax.nn.sigmoid(acc).astype(o_ref.dtype)


def kernel(x, w):
    N, C, H, W = x.shape
    HW = H * W
    xt = x.transpose(0, 2, 3, 1).reshape(N, HW, C)
    w_flat = w.reshape(-1).astype(jnp.float32)

    BN = 1
    for cand in (32, 16, 8, 4, 2, 1):
        if N % cand == 0:
            BN = cand
            break
    red_fn = functools.partial(_reduce_kernel, BN=BN, C=C, HW=HW)
    avg, mx = pl.pallas_call(
        red_fn,
        out_shape=(jax.ShapeDtypeStruct((N, HW), jnp.float32),
                   jax.ShapeDtypeStruct((N, HW), jnp.float32)),
        grid=(N // BN,),
        in_specs=[pl.BlockSpec((BN, HW, C), lambda i: (i, 0, 0))],
        out_specs=(pl.BlockSpec((BN, HW), lambda i: (i, 0)),
                   pl.BlockSpec((BN, HW), lambda i: (i, 0))),
        compiler_params=pltpu.CompilerParams(
            dimension_semantics=("parallel",),
            vmem_limit_bytes=48 * 1024 * 1024,
        ),
    )(xt)

    n_half = 2 if HW % 2 == 0 else 1
    HALF = HW // n_half
    PADS = _round_up(PAD * W + PAD, 8)
    rows = _round_up(HW + 2 * PADS, 8)
    conv_fn = functools.partial(_conv_kernel, N=N, H=H, W=W, HALF=HALF,
                                PADS=PADS)
    out_t = pl.pallas_call(
        conv_fn,
        out_shape=jax.ShapeDtypeStruct((HW, N), x.dtype),
        grid=(n_half,),
        in_specs=[
            pl.BlockSpec(memory_space=pltpu.MemorySpace.SMEM),
            pl.BlockSpec((N, HW), lambda i: (0, 0)),
            pl.BlockSpec((N, HW), lambda i: (0, 0)),
        ],
        out_specs=pl.BlockSpec((HALF, N), lambda i: (i, 0)),
        scratch_shapes=[
            pltpu.VMEM((rows, N), jnp.float32),
            pltpu.VMEM((rows, N), jnp.float32),
        ],
        compiler_params=pltpu.CompilerParams(
            dimension_semantics=("parallel",),
            vmem_limit_bytes=48 * 1024 * 1024,
        ),
    )(w_flat, avg, mx)

    return out_t.transpose(1, 0).reshape(N, 1, H, W)
